# baseline (speedup 1.0000x reference)
import functools

import jax
import jax.numpy as jnp
from jax import lax
from jax.experimental import pallas as pl
from jax.experimental.pallas import tpu as pltpu
from jax.experimental.pallas import tpu_sc as plsc

_N = 10000
_E = 320000
_D = 128
_NG = 64
_LAYERS = 8

_NC = 2
_NS = 16
_NW = _NC * _NS
_CH = 128
_NCHUNK = 80
_EPAD = _NW * _NCHUNK * _CH
_NPAD = _N + 8
_RPT = 632
_ZLAST = _NPAD - (_NS - 1) * _RPT
_WLAST = _N - (_NS - 1) * _RPT
_PCH = 80
_NPW = 320

_mesh = plsc.VectorSubcoreMesh(core_axis_name="c", subcore_axis_name="s")


def _acc_zero(zeros, acc, s):
    @pl.when(s < _NS - 1)
    def _():
        pltpu.sync_copy(zeros.at[pl.ds(s * _RPT, _RPT)],
                        acc.at[pl.ds(s * _RPT, _RPT)])

    @pl.when(s == _NS - 1)
    def _():
        pltpu.sync_copy(zeros.at[pl.ds((_NS - 1) * _RPT, _ZLAST)],
                        acc.at[pl.ds((_NS - 1) * _RPT, _ZLAST)])


def _acc_writeback(acc, out, c, s):
    @pl.when(s < _NS - 1)
    def _():
        pltpu.sync_copy(acc.at[pl.ds(s * _RPT, _RPT)],
                        out.at[c, pl.ds(s * _RPT, _RPT)])

    @pl.when(s == _NS - 1)
    def _():
        pltpu.sync_copy(acc.at[pl.ds((_NS - 1) * _RPT, _WLAST)],
                        out.at[c, pl.ds((_NS - 1) * _RPT, _WLAST)])


def _sc_segsum_body(h, src4, dst4, zeros, out,
                    dstb, sidx, rows, acc, semg, semsc, semi):
    c = lax.axis_index("c")
    s = lax.axis_index("s")
    _acc_zero(zeros, acc, s)
    pltpu.sync_copy(dst4.at[c, s], dstb)
    plsc.subcore_barrier()

    ngrp = _NCHUNK // 8

    def step(g, carry):
        gp = g % 2
        pltpu.sync_copy(src4.at[c, s, pl.ds(g * 8, 8)], sidx.at[gp])
        for j in range(8):
            pltpu.async_copy(h.at[sidx.at[gp, j]], rows.at[0],
                             semg.at[0]).wait()
            pltpu.sync_copy(rows.at[0], acc.at[dstb.at[g * 8 + j]], add=True)
        return carry

    lax.fori_loop(0, ngrp, step, 0)

    plsc.subcore_barrier()
    _acc_writeback(acc, out, c, s)


_sc_segsum = pl.kernel(
    _sc_segsum_body,
    out_type=jax.ShapeDtypeStruct((_NC, _N, _D), jnp.float32),
    mesh=_mesh,
    scratch_types=[
        pltpu.VMEM((_NCHUNK, _CH), jnp.int32),
        pltpu.VMEM((2, 8, _CH), jnp.int32),
        pltpu.VMEM((2, _CH, _D), jnp.float32),
        pltpu.VMEM_SHARED((_NPAD, _D), jnp.float32),
        pltpu.SemaphoreType.DMA((2,)),
        pltpu.SemaphoreType.DMA((2,)),
        pltpu.SemaphoreType.DMA((2,)),
    ],
)


def _sc_pool_body(xf, batch, zeros, ones2, ssum, cnt,
                  idx_b, rows, ones_v, sacc, cacc, sem):
    del sem
    c = lax.axis_index("c")
    s = lax.axis_index("s")
    wid = c * _NS + s

    @pl.when(s == 0)
    def _():
        pltpu.sync_copy(zeros.at[pl.ds(0, _NG)], sacc)
        pltpu.sync_copy(zeros.at[pl.ds(0, _NG)], cacc)

    pltpu.sync_copy(ones2, ones_v)
    plsc.subcore_barrier()
    base = wid * _NPW

    def chunk(t, carry):
        off = base + t * _PCH

        @pl.when(off < _N)
        def _():
            pltpu.sync_copy(batch.at[pl.ds(off, _PCH)], idx_b)
            pltpu.sync_copy(xf.at[pl.ds(off, _PCH)], rows)
            pltpu.sync_copy(rows, sacc.at[idx_b], add=True)
            pltpu.sync_copy(ones_v, cacc.at[idx_b], add=True)

        return carry

    lax.fori_loop(0, _NPW // _PCH, chunk, 0)
    plsc.subcore_barrier()

    @pl.when(s == 0)
    def _():
        pltpu.sync_copy(sacc, ssum.at[c])
        pltpu.sync_copy(cacc, cnt.at[c])


_sc_pool = pl.kernel(
    _sc_pool_body,
    out_type=[
        jax.ShapeDtypeStruct((_NC, _NG, _D), jnp.float32),
        jax.ShapeDtypeStruct((_NC, _NG, _D), jnp.float32),
    ],
    mesh=_mesh,
    scratch_types=[
        pltpu.VMEM((_PCH,), jnp.int32),
        pltpu.VMEM((_PCH, _D), jnp.float32),
        pltpu.VMEM((_PCH, _D), jnp.float32),
        pltpu.VMEM_SHARED((_NG, _D), jnp.float32),
        pltpu.VMEM_SHARED((_NG, _D), jnp.float32),
        pltpu.SemaphoreType.DMA,
    ],
)


def _ln_mm(x, g, b, wl, wr):
    mu = jnp.mean(x, axis=-1, keepdims=True)
    xc = x - mu
    var = jnp.mean(xc * xc, axis=-1, keepdims=True)
    ln = xc * lax.rsqrt(var + 1e-5) * g + b
    return (jnp.dot(ln, wl, preferred_element_type=jnp.float32),
            jnp.dot(ln, wr, preferred_element_type=jnp.float32))


def _tc_pre_body(x, g, b, wl, wr, xl_o, hr_o):
    xl, hr = _ln_mm(x[...], g[...], b[...], wl[...], wr[...])
    xl_o[...] = xl
    hr_o[...] = hr


_tc_pre = pl.pallas_call(
    _tc_pre_body,
    out_shape=(
        jax.ShapeDtypeStruct((_N, _D), jnp.float32),
        jax.ShapeDtypeStruct((_N, _D), jnp.float32),
    ),
)


def _agg_of(p, dg):
    d = dg[0, :, 0:1] + dg[1, :, 0:1]
    return (p[0] + p[1]) * (1.0 / jnp.maximum(d, 1.0))


def _tc_mid_body(save_x, has_res, *refs):
    (xl, p, dg, bvec, g2, b2, wl2, wr2), rest = refs[:8], refs[8:]
    if has_res:
        res, rest = rest[0], rest[1:]
    xn = jnp.maximum(xl[...] + _agg_of(p, dg) + bvec[...], 0.0)
    if has_res:
        xn = xn + res[...]
    if save_x:
        x_o, rest = rest[0], rest[1:]
        x_o[...] = xn
    xl_o, hr_o = rest
    xl2, hr2 = _ln_mm(xn, g2[...], b2[...], wl2[...], wr2[...])
    xl_o[...] = xl2
    hr_o[...] = hr2


def _make_mid(save_x, has_res):
    shapes = []
    if save_x:
        shapes.append(jax.ShapeDtypeStruct((_N, _D), jnp.float32))
    shapes.append(jax.ShapeDtypeStruct((_N, _D), jnp.float32))
    shapes.append(jax.ShapeDtypeStruct((_N, _D), jnp.float32))
    return pl.pallas_call(
        functools.partial(_tc_mid_body, save_x, has_res),
        out_shape=tuple(shapes),
    )


_tc_mid_plain = _make_mid(False, False)
_tc_mid_save = _make_mid(True, False)
_tc_mid_res = _make_mid(False, True)


def _tc_last_body(xl, p, dg, bvec, x_o):
    x_o[...] = jnp.maximum(xl[...] + _agg_of(p, dg) + bvec[...], 0.0)


_tc_last = pl.pallas_call(
    _tc_last_body,
    out_shape=jax.ShapeDtypeStruct((_N, _D), jnp.float32),
)


def _tc_out_body(sp, cp, w, b, o):
    pooled = (sp[0] + sp[1]) / jnp.maximum(cp[0] + cp[1], 1.0)
    o[...] = jnp.dot(pooled, w[...], preferred_element_type=jnp.float32) + b[...]


_tc_out = pl.pallas_call(
    _tc_out_body,
    out_shape=jax.ShapeDtypeStruct((_NG, _D), jnp.float32),
)


def kernel(x, edge_index, batch, params):
    npad = _EPAD - _E
    src_p = jnp.concatenate(
        [edge_index[0], jnp.zeros((npad,), jnp.int32)])
    dst_p = jnp.concatenate(
        [edge_index[1], _N + (jnp.arange(npad, dtype=jnp.int32) % 8)])
    src4 = src_p.reshape(_NC, _NS, _NCHUNK, _CH)
    dst4 = dst_p.reshape(_NC, _NS, _NCHUNK, _CH)
    zeros = jnp.zeros((_NPAD, _D), jnp.float32)
    ones2 = jnp.ones((_PCH, _D), jnp.float32)
    zerosg = jnp.zeros((_NG, _D), jnp.float32)
    ones_n = jnp.ones((_N, _D), jnp.float32)

    dg = _sc_segsum(ones_n, src4, dst4, zeros)
    xl, hr = _tc_pre(x, params["gamma0"], params["beta0"],
                     params["Wl0"], params["Wr0"])
    res4 = None
    for i in range(_LAYERS - 1):
        p = _sc_segsum(hr, src4, dst4, zeros)
        nxt = (params[f"gamma{i + 1}"], params[f"beta{i + 1}"],
               params[f"Wl{i + 1}"], params[f"Wr{i + 1}"])
        if i == 4:
            res4, xl, hr = _tc_mid_save(xl, p, dg, params[f"b{i}"], *nxt)
        elif i == 6:
            xl, hr = _tc_mid_res(xl, p, dg, params[f"b{i}"], *nxt, res4)
        else:
            xl, hr = _tc_mid_plain(xl, p, dg, params[f"b{i}"], *nxt)
    p = _sc_segsum(hr, src4, dst4, zeros)
    xf = _tc_last(xl, p, dg, params[f"b{_LAYERS - 1}"])
    ssum, cnt = _sc_pool(xf, batch, zerosg, ones2)
    return _tc_out(ssum, cnt, params["W_out"], params["b_out"])


# speedup vs baseline: 1.0937x; 1.0937x over previous
import functools

import jax
import jax.numpy as jnp
from jax import lax
from jax.experimental import pallas as pl
from jax.experimental.pallas import tpu as pltpu
from jax.experimental.pallas import tpu_sc as plsc

_N = 10000
_E = 320000
_D = 128
_NG = 64
_LAYERS = 8

_NC = 2
_NS = 16
_NW = _NC * _NS
_CH = 128
_NCHUNK = 80
_EPAD = _NW * _NCHUNK * _CH
_NPAD = _N + 8
_RPT = 632
_ZLAST = _NPAD - (_NS - 1) * _RPT
_WLAST = _N - (_NS - 1) * _RPT
_PCH = 80
_NPW = 320

_mesh = plsc.VectorSubcoreMesh(core_axis_name="c", subcore_axis_name="s")


def _acc_zero(zeros, acc, s):
    @pl.when(s < _NS - 1)
    def _():
        pltpu.sync_copy(zeros.at[pl.ds(s * _RPT, _RPT)],
                        acc.at[pl.ds(s * _RPT, _RPT)])

    @pl.when(s == _NS - 1)
    def _():
        pltpu.sync_copy(zeros.at[pl.ds((_NS - 1) * _RPT, _ZLAST)],
                        acc.at[pl.ds((_NS - 1) * _RPT, _ZLAST)])


def _acc_writeback(acc, out, c, s):
    @pl.when(s < _NS - 1)
    def _():
        pltpu.sync_copy(acc.at[pl.ds(s * _RPT, _RPT)],
                        out.at[c, pl.ds(s * _RPT, _RPT)])

    @pl.when(s == _NS - 1)
    def _():
        pltpu.sync_copy(acc.at[pl.ds((_NS - 1) * _RPT, _WLAST)],
                        out.at[c, pl.ds((_NS - 1) * _RPT, _WLAST)])


def _sc_segsum_body(h, src_p, dst_p, zeros, out, *sc):
    sidx = sc[0:8]
    didx = sc[8:16]
    rows = sc[16:18]
    acc = sc[18]
    semg = sc[19]
    semsc = sc[20]
    semi = sc[21:25]
    c = lax.axis_index("c")
    s = lax.axis_index("s")
    wid = c * _NS + s
    base = wid * (_NCHUNK * _CH)
    _acc_zero(zeros, acc, s)
    plsc.subcore_barrier()

    def idx_pair(q, m, k4):
        off = base + q * _CH
        a = pltpu.make_async_copy(src_p.at[pl.ds(off, _CH)], sidx[m],
                                  semi[k4])
        b = pltpu.make_async_copy(dst_p.at[pl.ds(off, _CH)], didx[m],
                                  semi[k4])
        return a, b

    def idx_issue(q, m, k4):
        off = base + q * _CH
        pltpu.async_copy(src_p.at[pl.ds(off, _CH)], sidx[m], semi[k4])
        pltpu.async_copy(dst_p.at[pl.ds(off, _CH)], didx[m], semi[k4])

    def gather(m, b):
        return pltpu.make_async_copy(h.at[sidx[m]], rows[b], semg)

    def scatter(m, b):
        return pltpu.make_async_copy(rows[b], acc.at[didx[m]], semsc)

    for q in range(4):
        idx_issue(q, q, q)

    def step(t, carry):
        for j in range(8):
            q = t * 8 + j
            m = j % 8
            b = j % 2
            k4 = j % 4
            m1 = (j - 1) % 8
            b1 = (j - 1) % 2
            ia, ib = idx_pair(q, m, k4)
            ia.wait()
            ib.wait()
            if j < 4:
                idx_issue(q + 4, (j + 4) % 8, k4)
            else:
                @pl.when(t < (_NCHUNK // 8) - 1)
                def _():
                    idx_issue(q + 4, (j + 4) % 8, k4)
            pltpu.async_copy(h.at[sidx[m]], rows[b], semg)
            if j == 0:
                @pl.when(t >= 1)
                def _():
                    scatter(m1, b1).wait()
            else:
                scatter(m1, b1).wait()
            gather(m, b).wait()
            pltpu.async_copy(rows[b], acc.at[didx[m]], semsc, add=True)
        return carry

    lax.fori_loop(0, _NCHUNK // 8, step, 0)
    scatter(7, 1).wait()

    plsc.subcore_barrier()
    _acc_writeback(acc, out, c, s)


_idx_scratch = [pltpu.VMEM((_CH,), jnp.int32) for _ in range(16)]
_sc_segsum = pl.kernel(
    _sc_segsum_body,
    out_type=jax.ShapeDtypeStruct((_NC, _N, _D), jnp.float32),
    mesh=_mesh,
    scratch_types=_idx_scratch + [
        pltpu.VMEM((_CH, _D), jnp.float32),
        pltpu.VMEM((_CH, _D), jnp.float32),
        pltpu.VMEM_SHARED((_NPAD, _D), jnp.float32),
        pltpu.SemaphoreType.DMA,
        pltpu.SemaphoreType.DMA,
        pltpu.SemaphoreType.DMA,
        pltpu.SemaphoreType.DMA,
        pltpu.SemaphoreType.DMA,
        pltpu.SemaphoreType.DMA,
    ],
)


def _sc_pool_body(xf, batch, zeros, ones2, ssum, cnt,
                  idx_b, rows, ones_v, sacc, cacc, sem):
    del sem
    c = lax.axis_index("c")
    s = lax.axis_index("s")
    wid = c * _NS + s

    @pl.when(s == 0)
    def _():
        pltpu.sync_copy(zeros.at[pl.ds(0, _NG)], sacc)
        pltpu.sync_copy(zeros.at[pl.ds(0, _NG)], cacc)

    pltpu.sync_copy(ones2, ones_v)
    plsc.subcore_barrier()
    base = wid * _NPW

    def chunk(t, carry):
        off = base + t * _PCH

        @pl.when(off < _N)
        def _():
            pltpu.sync_copy(batch.at[pl.ds(off, _PCH)], idx_b)
            pltpu.sync_copy(xf.at[pl.ds(off, _PCH)], rows)
            pltpu.sync_copy(rows, sacc.at[idx_b], add=True)
            pltpu.sync_copy(ones_v, cacc.at[idx_b], add=True)

        return carry

    lax.fori_loop(0, _NPW // _PCH, chunk, 0)
    plsc.subcore_barrier()

    @pl.when(s == 0)
    def _():
        pltpu.sync_copy(sacc, ssum.at[c])
        pltpu.sync_copy(cacc, cnt.at[c])


_sc_pool = pl.kernel(
    _sc_pool_body,
    out_type=[
        jax.ShapeDtypeStruct((_NC, _NG, _D), jnp.float32),
        jax.ShapeDtypeStruct((_NC, _NG, _D), jnp.float32),
    ],
    mesh=_mesh,
    scratch_types=[
        pltpu.VMEM((_PCH,), jnp.int32),
        pltpu.VMEM((_PCH, _D), jnp.float32),
        pltpu.VMEM((_PCH, _D), jnp.float32),
        pltpu.VMEM_SHARED((_NG, _D), jnp.float32),
        pltpu.VMEM_SHARED((_NG, _D), jnp.float32),
        pltpu.SemaphoreType.DMA,
    ],
)


def _ln_mm(x, g, b, wl, wr):
    mu = jnp.mean(x, axis=-1, keepdims=True)
    xc = x - mu
    var = jnp.mean(xc * xc, axis=-1, keepdims=True)
    ln = xc * lax.rsqrt(var + 1e-5) * g + b
    return (jnp.dot(ln, wl, preferred_element_type=jnp.float32),
            jnp.dot(ln, wr, preferred_element_type=jnp.float32))


def _tc_pre_body(x, g, b, wl, wr, xl_o, hr_o):
    xl, hr = _ln_mm(x[...], g[...], b[...], wl[...], wr[...])
    xl_o[...] = xl
    hr_o[...] = hr


_tc_pre = pl.pallas_call(
    _tc_pre_body,
    out_shape=(
        jax.ShapeDtypeStruct((_N, _D), jnp.float32),
        jax.ShapeDtypeStruct((_N, _D), jnp.float32),
    ),
)


def _agg_of(p, dg):
    d = dg[0, :, 0:1] + dg[1, :, 0:1]
    return (p[0] + p[1]) * (1.0 / jnp.maximum(d, 1.0))


def _tc_mid_body(save_x, has_res, *refs):
    (xl, p, dg, bvec, g2, b2, wl2, wr2), rest = refs[:8], refs[8:]
    if has_res:
        res, rest = rest[0], rest[1:]
    xn = jnp.maximum(xl[...] + _agg_of(p, dg) + bvec[...], 0.0)
    if has_res:
        xn = xn + res[...]
    if save_x:
        x_o, rest = rest[0], rest[1:]
        x_o[...] = xn
    xl_o, hr_o = rest
    xl2, hr2 = _ln_mm(xn, g2[...], b2[...], wl2[...], wr2[...])
    xl_o[...] = xl2
    hr_o[...] = hr2


def _make_mid(save_x, has_res):
    shapes = []
    if save_x:
        shapes.append(jax.ShapeDtypeStruct((_N, _D), jnp.float32))
    shapes.append(jax.ShapeDtypeStruct((_N, _D), jnp.float32))
    shapes.append(jax.ShapeDtypeStruct((_N, _D), jnp.float32))
    return pl.pallas_call(
        functools.partial(_tc_mid_body, save_x, has_res),
        out_shape=tuple(shapes),
    )


_tc_mid_plain = _make_mid(False, False)
_tc_mid_save = _make_mid(True, False)
_tc_mid_res = _make_mid(False, True)


def _tc_last_body(xl, p, dg, bvec, x_o):
    x_o[...] = jnp.maximum(xl[...] + _agg_of(p, dg) + bvec[...], 0.0)


_tc_last = pl.pallas_call(
    _tc_last_body,
    out_shape=jax.ShapeDtypeStruct((_N, _D), jnp.float32),
)


def _tc_out_body(sp, cp, w, b, o):
    pooled = (sp[0] + sp[1]) / jnp.maximum(cp[0] + cp[1], 1.0)
    o[...] = jnp.dot(pooled, w[...], preferred_element_type=jnp.float32) + b[...]


_tc_out = pl.pallas_call(
    _tc_out_body,
    out_shape=jax.ShapeDtypeStruct((_NG, _D), jnp.float32),
)


def kernel(x, edge_index, batch, params):
    npad = _EPAD - _E
    src_p = jnp.concatenate(
        [edge_index[0], jnp.zeros((npad,), jnp.int32)])
    dst_p = jnp.concatenate(
        [edge_index[1], _N + (jnp.arange(npad, dtype=jnp.int32) % 8)])
    zeros = jnp.zeros((_NPAD, _D), jnp.float32)
    ones2 = jnp.ones((_PCH, _D), jnp.float32)
    zerosg = jnp.zeros((_NG, _D), jnp.float32)
    ones_n = jnp.ones((_N, _D), jnp.float32)

    dg = _sc_segsum(ones_n, src_p, dst_p, zeros)
    xl, hr = _tc_pre(x, params["gamma0"], params["beta0"],
                     params["Wl0"], params["Wr0"])
    res4 = None
    for i in range(_LAYERS - 1):
        p = _sc_segsum(hr, src_p, dst_p, zeros)
        nxt = (params[f"gamma{i + 1}"], params[f"beta{i + 1}"],
               params[f"Wl{i + 1}"], params[f"Wr{i + 1}"])
        if i == 4:
            res4, xl, hr = _tc_mid_save(xl, p, dg, params[f"b{i}"], *nxt)
        elif i == 6:
            xl, hr = _tc_mid_res(xl, p, dg, params[f"b{i}"], *nxt, res4)
        else:
            xl, hr = _tc_mid_plain(xl, p, dg, params[f"b{i}"], *nxt)
    p = _sc_segsum(hr, src_p, dst_p, zeros)
    xf = _tc_last(xl, p, dg, params[f"b{_LAYERS - 1}"])
    ssum, cnt = _sc_pool(xf, batch, zerosg, ones2)
    return _tc_out(ssum, cnt, params["W_out"], params["b_out"])


# speedup vs baseline: 3.4783x; 3.1802x over previous
import functools

import jax
import jax.numpy as jnp
from jax import lax
from jax.experimental import pallas as pl
from jax.experimental.pallas import tpu as pltpu
from jax.experimental.pallas import tpu_sc as plsc

_N = 10000
_E = 320000
_D = 128
_NG = 64
_LAYERS = 8

_NC = 2
_NS = 16
_NW = _NC * _NS
_CH = 128
_NCHUNK = 80
_EPAD = _NW * _NCHUNK * _CH
_NPAD = _N + 120
_RPT = 632
_ZLAST = _NPAD - (_NS - 1) * _RPT
_WLAST = _N - (_NS - 1) * _RPT
_PCH = 80
_NPW = 320

_mesh = plsc.VectorSubcoreMesh(core_axis_name="c", subcore_axis_name="s")


def _acc_zero(zeros, acc, s):
    @pl.when(s < _NS - 1)
    def _():
        pltpu.sync_copy(zeros.at[pl.ds(s * _RPT, _RPT)],
                        acc.at[pl.ds(s * _RPT, _RPT)])

    @pl.when(s == _NS - 1)
    def _():
        pltpu.sync_copy(zeros.at[pl.ds((_NS - 1) * _RPT, _ZLAST)],
                        acc.at[pl.ds((_NS - 1) * _RPT, _ZLAST)])


def _acc_writeback(acc, out, c, s):
    @pl.when(s < _NS - 1)
    def _():
        pltpu.sync_copy(acc.at[pl.ds(s * _RPT, _RPT)],
                        out.at[c, pl.ds(s * _RPT, _RPT)])

    @pl.when(s == _NS - 1)
    def _():
        pltpu.sync_copy(acc.at[pl.ds((_NS - 1) * _RPT, _WLAST)],
                        out.at[c, pl.ds((_NS - 1) * _RPT, _WLAST)])


def _sc_segsum_body(h, src_p, dst_p, zeros, out, *sc):
    sidx = sc[0:8]
    didx = sc[8:16]
    rows = sc[16:18]
    acc = sc[18]
    semg = sc[19]
    semsc = sc[20]
    semi = sc[21:25]
    c = lax.axis_index("c")
    s = lax.axis_index("s")
    wid = c * _NS + s
    base = wid * (_NCHUNK * _CH)
    _acc_zero(zeros, acc, s)
    plsc.subcore_barrier()

    def idx_pair(q, m, k4):
        off = base + q * _CH
        a = pltpu.make_async_copy(src_p.at[pl.ds(off, _CH)], sidx[m],
                                  semi[k4])
        b = pltpu.make_async_copy(dst_p.at[pl.ds(off, _CH)], didx[m],
                                  semi[k4])
        return a, b

    def idx_issue(q, m, k4):
        off = base + q * _CH
        pltpu.async_copy(src_p.at[pl.ds(off, _CH)], sidx[m], semi[k4])
        pltpu.async_copy(dst_p.at[pl.ds(off, _CH)], didx[m], semi[k4])

    def gather(m, b):
        return pltpu.make_async_copy(h.at[sidx[m]], rows[b], semg)

    def scatter(m, b):
        return pltpu.make_async_copy(rows[b], acc.at[didx[m]], semsc)

    for q in range(4):
        idx_issue(q, q, q)

    def step(t, carry):
        for j in range(8):
            q = t * 8 + j
            m = j % 8
            b = j % 2
            k4 = j % 4
            m1 = (j - 1) % 8
            b1 = (j - 1) % 2
            ia, ib = idx_pair(q, m, k4)
            ia.wait()
            ib.wait()
            if j < 4:
                idx_issue(q + 4, (j + 4) % 8, k4)
            else:
                @pl.when(t < (_NCHUNK // 8) - 1)
                def _():
                    idx_issue(q + 4, (j + 4) % 8, k4)
            pltpu.async_copy(h.at[sidx[m]], rows[b], semg)
            if j == 0:
                @pl.when(t >= 1)
                def _():
                    scatter(m1, b1).wait()
            else:
                scatter(m1, b1).wait()
            gather(m, b).wait()
            pltpu.async_copy(rows[b], acc.at[didx[m]], semsc, add=True)
        return carry

    lax.fori_loop(0, _NCHUNK // 8, step, 0)
    scatter(7, 1).wait()

    plsc.subcore_barrier()
    _acc_writeback(acc, out, c, s)


_idx_scratch = [pltpu.VMEM((_CH,), jnp.int32) for _ in range(16)]
_sc_segsum = pl.kernel(
    _sc_segsum_body,
    out_type=jax.ShapeDtypeStruct((_NC, _N, _D), jnp.float32),
    mesh=_mesh,
    scratch_types=_idx_scratch + [
        pltpu.VMEM((_CH, _D), jnp.float32),
        pltpu.VMEM((_CH, _D), jnp.float32),
        pltpu.VMEM_SHARED((_NPAD, _D), jnp.float32),
        pltpu.SemaphoreType.DMA,
        pltpu.SemaphoreType.DMA,
        pltpu.SemaphoreType.DMA,
        pltpu.SemaphoreType.DMA,
        pltpu.SemaphoreType.DMA,
        pltpu.SemaphoreType.DMA,
    ],
)


def _sc_pool_body(xf, batch, zeros, ones2, ssum, cnt,
                  idx_b, rows, ones_v, sacc, cacc, sem):
    del sem
    c = lax.axis_index("c")
    s = lax.axis_index("s")
    wid = c * _NS + s

    @pl.when(s == 0)
    def _():
        pltpu.sync_copy(zeros.at[pl.ds(0, _NG)], sacc)
        pltpu.sync_copy(zeros.at[pl.ds(0, _NG)], cacc)

    pltpu.sync_copy(ones2, ones_v)
    plsc.subcore_barrier()
    base = wid * _NPW

    def chunk(t, carry):
        off = base + t * _PCH

        @pl.when(off < _N)
        def _():
            pltpu.sync_copy(batch.at[pl.ds(off, _PCH)], idx_b)
            pltpu.sync_copy(xf.at[pl.ds(off, _PCH)], rows)
            pltpu.sync_copy(rows, sacc.at[idx_b], add=True)
            pltpu.sync_copy(ones_v, cacc.at[idx_b], add=True)

        return carry

    lax.fori_loop(0, _NPW // _PCH, chunk, 0)
    plsc.subcore_barrier()

    @pl.when(s == 0)
    def _():
        pltpu.sync_copy(sacc, ssum.at[c])
        pltpu.sync_copy(cacc, cnt.at[c])


_sc_pool = pl.kernel(
    _sc_pool_body,
    out_type=[
        jax.ShapeDtypeStruct((_NC, _NG, _D), jnp.float32),
        jax.ShapeDtypeStruct((_NC, _NG, _D), jnp.float32),
    ],
    mesh=_mesh,
    scratch_types=[
        pltpu.VMEM((_PCH,), jnp.int32),
        pltpu.VMEM((_PCH, _D), jnp.float32),
        pltpu.VMEM((_PCH, _D), jnp.float32),
        pltpu.VMEM_SHARED((_NG, _D), jnp.float32),
        pltpu.VMEM_SHARED((_NG, _D), jnp.float32),
        pltpu.SemaphoreType.DMA,
    ],
)


def _ln_mm(x, g, b, wl, wr):
    mu = jnp.mean(x, axis=-1, keepdims=True)
    xc = x - mu
    var = jnp.mean(xc * xc, axis=-1, keepdims=True)
    ln = xc * lax.rsqrt(var + 1e-5) * g + b
    return (jnp.dot(ln, wl, preferred_element_type=jnp.float32),
            jnp.dot(ln, wr, preferred_element_type=jnp.float32))


def _tc_pre_body(x, g, b, wl, wr, xl_o, hr_o):
    xl, hr = _ln_mm(x[...], g[...], b[...], wl[...], wr[...])
    xl_o[...] = xl
    hr_o[...] = hr


_tc_pre = pl.pallas_call(
    _tc_pre_body,
    out_shape=(
        jax.ShapeDtypeStruct((_N, _D), jnp.float32),
        jax.ShapeDtypeStruct((_N, _D), jnp.float32),
    ),
)


def _agg_of(p, dg):
    d = dg[0, :, 0:1] + dg[1, :, 0:1]
    return (p[0] + p[1]) * (1.0 / jnp.maximum(d, 1.0))


def _tc_mid_body(save_x, has_res, *refs):
    (xl, p, dg, bvec, g2, b2, wl2, wr2), rest = refs[:8], refs[8:]
    if has_res:
        res, rest = rest[0], rest[1:]
    xn = jnp.maximum(xl[...] + _agg_of(p, dg) + bvec[...], 0.0)
    if has_res:
        xn = xn + res[...]
    if save_x:
        x_o, rest = rest[0], rest[1:]
        x_o[...] = xn
    xl_o, hr_o = rest
    xl2, hr2 = _ln_mm(xn, g2[...], b2[...], wl2[...], wr2[...])
    xl_o[...] = xl2
    hr_o[...] = hr2


def _make_mid(save_x, has_res):
    shapes = []
    if save_x:
        shapes.append(jax.ShapeDtypeStruct((_N, _D), jnp.float32))
    shapes.append(jax.ShapeDtypeStruct((_N, _D), jnp.float32))
    shapes.append(jax.ShapeDtypeStruct((_N, _D), jnp.float32))
    return pl.pallas_call(
        functools.partial(_tc_mid_body, save_x, has_res),
        out_shape=tuple(shapes),
    )


_tc_mid_plain = _make_mid(False, False)
_tc_mid_save = _make_mid(True, False)
_tc_mid_res = _make_mid(False, True)


def _tc_last_body(xl, p, dg, bvec, x_o):
    x_o[...] = jnp.maximum(xl[...] + _agg_of(p, dg) + bvec[...], 0.0)


_tc_last = pl.pallas_call(
    _tc_last_body,
    out_shape=jax.ShapeDtypeStruct((_N, _D), jnp.float32),
)


def _tc_out_body(sp, cp, w, b, o):
    pooled = (sp[0] + sp[1]) / jnp.maximum(cp[0] + cp[1], 1.0)
    o[...] = jnp.dot(pooled, w[...], preferred_element_type=jnp.float32) + b[...]


_tc_out = pl.pallas_call(
    _tc_out_body,
    out_shape=jax.ShapeDtypeStruct((_NG, _D), jnp.float32),
)


def kernel(x, edge_index, batch, params):
    ppw = (_EPAD - _E) // _NW
    spad = jnp.broadcast_to(jnp.arange(ppw, dtype=jnp.int32) % 128,
                            (_NW, ppw))
    dpad = jnp.broadcast_to(_N + (jnp.arange(ppw, dtype=jnp.int32) % 120),
                            (_NW, ppw))
    src_p = jnp.concatenate(
        [edge_index[0].reshape(_NW, _E // _NW), spad], axis=1).reshape(-1)
    dst_p = jnp.concatenate(
        [edge_index[1].reshape(_NW, _E // _NW), dpad], axis=1).reshape(-1)
    zeros = jnp.zeros((_NPAD, _D), jnp.float32)
    ones2 = jnp.ones((_PCH, _D), jnp.float32)
    zerosg = jnp.zeros((_NG, _D), jnp.float32)
    ones_n = jnp.ones((_N, _D), jnp.float32)

    dg = _sc_segsum(ones_n, src_p, dst_p, zeros)
    xl, hr = _tc_pre(x, params["gamma0"], params["beta0"],
                     params["Wl0"], params["Wr0"])
    res4 = None
    for i in range(_LAYERS - 1):
        p = _sc_segsum(hr, src_p, dst_p, zeros)
        nxt = (params[f"gamma{i + 1}"], params[f"beta{i + 1}"],
               params[f"Wl{i + 1}"], params[f"Wr{i + 1}"])
        if i == 4:
            res4, xl, hr = _tc_mid_save(xl, p, dg, params[f"b{i}"], *nxt)
        elif i == 6:
            xl, hr = _tc_mid_res(xl, p, dg, params[f"b{i}"], *nxt, res4)
        else:
            xl, hr = _tc_mid_plain(xl, p, dg, params[f"b{i}"], *nxt)
    p = _sc_segsum(hr, src_p, dst_p, zeros)
    xf = _tc_last(xl, p, dg, params[f"b{_LAYERS - 1}"])
    ssum, cnt = _sc_pool(xf, batch, zerosg, ones2)
    return _tc_out(ssum, cnt, params["W_out"], params["b_out"])


# speedup vs baseline: 4.0750x; 1.1716x over previous
import functools

import jax
import jax.numpy as jnp
from jax import lax
from jax.experimental import pallas as pl
from jax.experimental.pallas import tpu as pltpu
from jax.experimental.pallas import tpu_sc as plsc

_N = 10000
_E = 320000
_D = 128
_NG = 64
_LAYERS = 8

_NC = 2
_NS = 16
_NW = _NC * _NS
_CH = 128
_NCHUNK = 80
_EPAD = _NW * _NCHUNK * _CH
_NPAD = _N + 120
_RPT = 632
_ZLAST = _NPAD - (_NS - 1) * _RPT
_WLAST = _N - (_NS - 1) * _RPT
_PCH = 80
_NPW = 320

_mesh = plsc.VectorSubcoreMesh(core_axis_name="c", subcore_axis_name="s")


def _acc_zero(zeros, acc, s):
    @pl.when(s < _NS - 1)
    def _():
        pltpu.sync_copy(zeros.at[pl.ds(s * _RPT, _RPT)],
                        acc.at[pl.ds(s * _RPT, _RPT)])

    @pl.when(s == _NS - 1)
    def _():
        pltpu.sync_copy(zeros.at[pl.ds((_NS - 1) * _RPT, _ZLAST)],
                        acc.at[pl.ds((_NS - 1) * _RPT, _ZLAST)])


def _acc_writeback(acc, out, c, s):
    @pl.when(s < _NS - 1)
    def _():
        pltpu.sync_copy(acc.at[pl.ds(s * _RPT, _RPT)],
                        out.at[c, pl.ds(s * _RPT, _RPT)])

    @pl.when(s == _NS - 1)
    def _():
        pltpu.sync_copy(acc.at[pl.ds((_NS - 1) * _RPT, _WLAST)],
                        out.at[c, pl.ds((_NS - 1) * _RPT, _WLAST)])


def _sc_segsum_body(h, src_p, dst_p, zeros, out, *sc):
    sidx = sc[0:8]
    didx = sc[8:16]
    rows = sc[16:18]
    acc = sc[18]
    semg = sc[19:21]
    semsc = sc[21]
    semi = sc[22:26]
    c = lax.axis_index("c")
    s = lax.axis_index("s")
    wid = c * _NS + s
    base = wid * (_NCHUNK * _CH)
    _acc_zero(zeros, acc, s)
    plsc.subcore_barrier()

    def idx_pair(q, m, k4):
        off = base + q * _CH
        a = pltpu.make_async_copy(src_p.at[pl.ds(off, _CH)], sidx[m],
                                  semi[k4])
        b = pltpu.make_async_copy(dst_p.at[pl.ds(off, _CH)], didx[m],
                                  semi[k4])
        return a, b

    def idx_issue(q, m, k4):
        off = base + q * _CH
        pltpu.async_copy(src_p.at[pl.ds(off, _CH)], sidx[m], semi[k4])
        pltpu.async_copy(dst_p.at[pl.ds(off, _CH)], didx[m], semi[k4])

    def gather(m, b):
        return pltpu.make_async_copy(h.at[sidx[m]], rows[b], semg[b])

    def scatter(m, b):
        return pltpu.make_async_copy(rows[b], acc.at[didx[m]], semsc)

    for q in range(4):
        idx_issue(q, q, q)
    ia0, ib0 = idx_pair(0, 0, 0)
    ia0.wait()
    ib0.wait()
    pltpu.async_copy(h.at[sidx[0]], rows[0], semg[0])

    last = _NCHUNK // 8 - 1

    def step(t, carry):
        for j in range(8):
            q = t * 8 + j
            m = j % 8
            b = j % 2
            k4 = j % 4
            m1 = (j + 1) % 8
            b1 = (j + 1) % 2
            if j == 7:
                @pl.when(t < last)
                def _():
                    ia, ib = idx_pair(q + 1, m1, (j + 1) % 4)
                    ia.wait()
                    ib.wait()
                    idx_issue(q + 4, (j + 4) % 8, k4)
            else:
                ia, ib = idx_pair(q + 1, m1, (j + 1) % 4)
                ia.wait()
                ib.wait()
                if j < 4:
                    idx_issue(q + 4, (j + 4) % 8, k4)
                else:
                    @pl.when(t < last)
                    def _():
                        idx_issue(q + 4, (j + 4) % 8, k4)
            if j == 0:
                @pl.when(t >= 1)
                def _():
                    scatter((j - 1) % 8, (j - 1) % 2).wait()
            else:
                scatter((j - 1) % 8, (j - 1) % 2).wait()
            if j == 7:
                @pl.when(t < last)
                def _():
                    pltpu.async_copy(h.at[sidx[m1]], rows[b1], semg[b1])
            else:
                pltpu.async_copy(h.at[sidx[m1]], rows[b1], semg[b1])
            gather(m, b).wait()
            pltpu.async_copy(rows[b], acc.at[didx[m]], semsc, add=True)
        return carry

    lax.fori_loop(0, _NCHUNK // 8, step, 0)
    scatter(7, 1).wait()

    plsc.subcore_barrier()
    _acc_writeback(acc, out, c, s)


_idx_scratch = [pltpu.VMEM((_CH,), jnp.int32) for _ in range(16)]
_sc_segsum = pl.kernel(
    _sc_segsum_body,
    out_type=jax.ShapeDtypeStruct((_NC, _N, _D), jnp.float32),
    mesh=_mesh,
    scratch_types=_idx_scratch + [
        pltpu.VMEM((_CH, _D), jnp.float32),
        pltpu.VMEM((_CH, _D), jnp.float32),
        pltpu.VMEM_SHARED((_NPAD, _D), jnp.float32),
        pltpu.SemaphoreType.DMA,
        pltpu.SemaphoreType.DMA,
        pltpu.SemaphoreType.DMA,
        pltpu.SemaphoreType.DMA,
        pltpu.SemaphoreType.DMA,
        pltpu.SemaphoreType.DMA,
        pltpu.SemaphoreType.DMA,
    ],
)


def _sc_pool_body(xf, batch, zeros, ones2, ssum, cnt,
                  idx_b, rows, ones_v, sacc, cacc, sem):
    del sem
    c = lax.axis_index("c")
    s = lax.axis_index("s")
    wid = c * _NS + s

    @pl.when(s == 0)
    def _():
        pltpu.sync_copy(zeros.at[pl.ds(0, _NG)], sacc)
        pltpu.sync_copy(zeros.at[pl.ds(0, _NG)], cacc)

    pltpu.sync_copy(ones2, ones_v)
    plsc.subcore_barrier()
    base = wid * _NPW

    def chunk(t, carry):
        off = base + t * _PCH

        @pl.when(off < _N)
        def _():
            pltpu.sync_copy(batch.at[pl.ds(off, _PCH)], idx_b)
            pltpu.sync_copy(xf.at[pl.ds(off, _PCH)], rows)
            pltpu.sync_copy(rows, sacc.at[idx_b], add=True)
            pltpu.sync_copy(ones_v, cacc.at[idx_b], add=True)

        return carry

    lax.fori_loop(0, _NPW // _PCH, chunk, 0)
    plsc.subcore_barrier()

    @pl.when(s == 0)
    def _():
        pltpu.sync_copy(sacc, ssum.at[c])
        pltpu.sync_copy(cacc, cnt.at[c])


_sc_pool = pl.kernel(
    _sc_pool_body,
    out_type=[
        jax.ShapeDtypeStruct((_NC, _NG, _D), jnp.float32),
        jax.ShapeDtypeStruct((_NC, _NG, _D), jnp.float32),
    ],
    mesh=_mesh,
    scratch_types=[
        pltpu.VMEM((_PCH,), jnp.int32),
        pltpu.VMEM((_PCH, _D), jnp.float32),
        pltpu.VMEM((_PCH, _D), jnp.float32),
        pltpu.VMEM_SHARED((_NG, _D), jnp.float32),
        pltpu.VMEM_SHARED((_NG, _D), jnp.float32),
        pltpu.SemaphoreType.DMA,
    ],
)


def _ln_mm(x, g, b, wl, wr):
    mu = jnp.mean(x, axis=-1, keepdims=True)
    xc = x - mu
    var = jnp.mean(xc * xc, axis=-1, keepdims=True)
    ln = xc * lax.rsqrt(var + 1e-5) * g + b
    return (jnp.dot(ln, wl, preferred_element_type=jnp.float32),
            jnp.dot(ln, wr, preferred_element_type=jnp.float32))


def _tc_pre_body(x, g, b, wl, wr, xl_o, hr_o):
    xl, hr = _ln_mm(x[...], g[...], b[...], wl[...], wr[...])
    xl_o[...] = xl
    hr_o[...] = hr


_tc_pre = pl.pallas_call(
    _tc_pre_body,
    out_shape=(
        jax.ShapeDtypeStruct((_N, _D), jnp.float32),
        jax.ShapeDtypeStruct((_N, _D), jnp.float32),
    ),
)


def _agg_of(p, dg):
    d = dg[0, :, 0:1] + dg[1, :, 0:1]
    return (p[0] + p[1]) * (1.0 / jnp.maximum(d, 1.0))


def _tc_mid_body(save_x, has_res, *refs):
    (xl, p, dg, bvec, g2, b2, wl2, wr2), rest = refs[:8], refs[8:]
    if has_res:
        res, rest = rest[0], rest[1:]
    xn = jnp.maximum(xl[...] + _agg_of(p, dg) + bvec[...], 0.0)
    if has_res:
        xn = xn + res[...]
    if save_x:
        x_o, rest = rest[0], rest[1:]
        x_o[...] = xn
    xl_o, hr_o = rest
    xl2, hr2 = _ln_mm(xn, g2[...], b2[...], wl2[...], wr2[...])
    xl_o[...] = xl2
    hr_o[...] = hr2


def _make_mid(save_x, has_res):
    shapes = []
    if save_x:
        shapes.append(jax.ShapeDtypeStruct((_N, _D), jnp.float32))
    shapes.append(jax.ShapeDtypeStruct((_N, _D), jnp.float32))
    shapes.append(jax.ShapeDtypeStruct((_N, _D), jnp.float32))
    return pl.pallas_call(
        functools.partial(_tc_mid_body, save_x, has_res),
        out_shape=tuple(shapes),
    )


_tc_mid_plain = _make_mid(False, False)
_tc_mid_save = _make_mid(True, False)
_tc_mid_res = _make_mid(False, True)


def _tc_last_body(xl, p, dg, bvec, x_o):
    x_o[...] = jnp.maximum(xl[...] + _agg_of(p, dg) + bvec[...], 0.0)


_tc_last = pl.pallas_call(
    _tc_last_body,
    out_shape=jax.ShapeDtypeStruct((_N, _D), jnp.float32),
)


def _tc_out_body(sp, cp, w, b, o):
    pooled = (sp[0] + sp[1]) / jnp.maximum(cp[0] + cp[1], 1.0)
    o[...] = jnp.dot(pooled, w[...], preferred_element_type=jnp.float32) + b[...]


_tc_out = pl.pallas_call(
    _tc_out_body,
    out_shape=jax.ShapeDtypeStruct((_NG, _D), jnp.float32),
)


def kernel(x, edge_index, batch, params):
    ppw = (_EPAD - _E) // _NW
    spad = jnp.broadcast_to(jnp.arange(ppw, dtype=jnp.int32) % 128,
                            (_NW, ppw))
    dpad = jnp.broadcast_to(_N + (jnp.arange(ppw, dtype=jnp.int32) % 120),
                            (_NW, ppw))
    src_p = jnp.concatenate(
        [edge_index[0].reshape(_NW, _E // _NW), spad], axis=1).reshape(-1)
    dst_p = jnp.concatenate(
        [edge_index[1].reshape(_NW, _E // _NW), dpad], axis=1).reshape(-1)
    zeros = jnp.zeros((_NPAD, _D), jnp.float32)
    ones2 = jnp.ones((_PCH, _D), jnp.float32)
    zerosg = jnp.zeros((_NG, _D), jnp.float32)
    ones_n = jnp.ones((_N, _D), jnp.float32)

    dg = _sc_segsum(ones_n, src_p, dst_p, zeros)
    xl, hr = _tc_pre(x, params["gamma0"], params["beta0"],
                     params["Wl0"], params["Wr0"])
    res4 = None
    for i in range(_LAYERS - 1):
        p = _sc_segsum(hr, src_p, dst_p, zeros)
        nxt = (params[f"gamma{i + 1}"], params[f"beta{i + 1}"],
               params[f"Wl{i + 1}"], params[f"Wr{i + 1}"])
        if i == 4:
            res4, xl, hr = _tc_mid_save(xl, p, dg, params[f"b{i}"], *nxt)
        elif i == 6:
            xl, hr = _tc_mid_res(xl, p, dg, params[f"b{i}"], *nxt, res4)
        else:
            xl, hr = _tc_mid_plain(xl, p, dg, params[f"b{i}"], *nxt)
    p = _sc_segsum(hr, src_p, dst_p, zeros)
    xf = _tc_last(xl, p, dg, params[f"b{_LAYERS - 1}"])
    ssum, cnt = _sc_pool(xf, batch, zerosg, ones2)
    return _tc_out(ssum, cnt, params["W_out"], params["b_out"])


# speedup vs baseline: 4.1306x; 1.0137x over previous
import functools

import jax
import jax.numpy as jnp
from jax import lax
from jax.experimental import pallas as pl
from jax.experimental.pallas import tpu as pltpu
from jax.experimental.pallas import tpu_sc as plsc

_N = 10000
_E = 320000
_D = 128
_NG = 64
_LAYERS = 8

_NC = 2
_NS = 16
_NW = _NC * _NS
_CH = 128
_NCHUNK = 80
_EPAD = _NW * _NCHUNK * _CH
_NPAD = _N + 120
_RPT = 632
_ZLAST = _NPAD - (_NS - 1) * _RPT
_WLAST = _N - (_NS - 1) * _RPT
_PCH = 80
_NPW = 320

_mesh = plsc.VectorSubcoreMesh(core_axis_name="c", subcore_axis_name="s")


def _acc_zero(zeros, acc, s):
    @pl.when(s < _NS - 1)
    def _():
        pltpu.sync_copy(zeros.at[pl.ds(s * _RPT, _RPT)],
                        acc.at[pl.ds(s * _RPT, _RPT)])

    @pl.when(s == _NS - 1)
    def _():
        pltpu.sync_copy(zeros.at[pl.ds((_NS - 1) * _RPT, _ZLAST)],
                        acc.at[pl.ds((_NS - 1) * _RPT, _ZLAST)])


def _acc_writeback(acc, out, c, s):
    @pl.when(s < _NS - 1)
    def _():
        pltpu.sync_copy(acc.at[pl.ds(s * _RPT, _RPT)],
                        out.at[c, pl.ds(s * _RPT, _RPT)])

    @pl.when(s == _NS - 1)
    def _():
        pltpu.sync_copy(acc.at[pl.ds((_NS - 1) * _RPT, _WLAST)],
                        out.at[c, pl.ds((_NS - 1) * _RPT, _WLAST)])


def _sc_segsum_body(h, src_p, dst_p, zeros, out, *sc):
    sidx = sc[0:8]
    didx = sc[8:16]
    rows = sc[16:18]
    acc = sc[18]
    semg = sc[19:21]
    semsc = sc[21]
    semi = sc[22:26]
    c = lax.axis_index("c")
    s = lax.axis_index("s")
    wid = c * _NS + s
    base = wid * (_NCHUNK * _CH)
    _acc_zero(zeros, acc, s)
    plsc.subcore_barrier()

    def idx_pair(q, m, k4):
        off = base + q * _CH
        a = pltpu.make_async_copy(src_p.at[pl.ds(off, _CH)], sidx[m],
                                  semi[k4])
        b = pltpu.make_async_copy(dst_p.at[pl.ds(off, _CH)], didx[m],
                                  semi[k4])
        return a, b

    def idx_issue(q, m, k4):
        off = base + q * _CH
        pltpu.async_copy(src_p.at[pl.ds(off, _CH)], sidx[m], semi[k4])
        pltpu.async_copy(dst_p.at[pl.ds(off, _CH)], didx[m], semi[k4])

    def gather(m, b):
        return pltpu.make_async_copy(h.at[sidx[m]], rows[b], semg[b])

    def scatter(m, b):
        return pltpu.make_async_copy(rows[b], acc.at[didx[m]], semsc)

    for q in range(4):
        idx_issue(q, q, q)
    ia0, ib0 = idx_pair(0, 0, 0)
    ia0.wait()
    ib0.wait()
    pltpu.async_copy(h.at[sidx[0]], rows[0], semg[0])

    last = _NCHUNK // 8 - 1

    def step(t, carry):
        for j in range(8):
            q = t * 8 + j
            m = j % 8
            b = j % 2
            k4 = j % 4
            m1 = (j + 1) % 8
            b1 = (j + 1) % 2
            if j == 7:
                @pl.when(t < last)
                def _():
                    ia, ib = idx_pair(q + 1, m1, (j + 1) % 4)
                    ia.wait()
                    ib.wait()
                    idx_issue(q + 4, (j + 4) % 8, k4)
            else:
                ia, ib = idx_pair(q + 1, m1, (j + 1) % 4)
                ia.wait()
                ib.wait()
                if j < 4:
                    idx_issue(q + 4, (j + 4) % 8, k4)
                else:
                    @pl.when(t < last)
                    def _():
                        idx_issue(q + 4, (j + 4) % 8, k4)
            if j == 0:
                @pl.when(t >= 1)
                def _():
                    scatter((j - 1) % 8, (j - 1) % 2).wait()
            else:
                scatter((j - 1) % 8, (j - 1) % 2).wait()
            if j == 7:
                @pl.when(t < last)
                def _():
                    pltpu.async_copy(h.at[sidx[m1]], rows[b1], semg[b1])
            else:
                pltpu.async_copy(h.at[sidx[m1]], rows[b1], semg[b1])
            gather(m, b).wait()
            pltpu.async_copy(rows[b], acc.at[didx[m]], semsc, add=True)
        return carry

    lax.fori_loop(0, _NCHUNK // 8, step, 0)
    scatter(7, 1).wait()

    plsc.subcore_barrier()
    _acc_writeback(acc, out, c, s)


_idx_scratch = [pltpu.VMEM((_CH,), jnp.int32) for _ in range(16)]
_sc_segsum = pl.kernel(
    _sc_segsum_body,
    out_type=jax.ShapeDtypeStruct((_NC, _N, _D), jnp.float32),
    mesh=_mesh,
    scratch_types=_idx_scratch + [
        pltpu.VMEM((_CH, _D), jnp.float32),
        pltpu.VMEM((_CH, _D), jnp.float32),
        pltpu.VMEM_SHARED((_NPAD, _D), jnp.float32),
        pltpu.SemaphoreType.DMA,
        pltpu.SemaphoreType.DMA,
        pltpu.SemaphoreType.DMA,
        pltpu.SemaphoreType.DMA,
        pltpu.SemaphoreType.DMA,
        pltpu.SemaphoreType.DMA,
        pltpu.SemaphoreType.DMA,
    ],
)


def _sc_pool_body(xf, batch, zeros, ones2, ssum, cnt,
                  idx_b, rows, ones_v, sacc, cacc, sem):
    del sem
    c = lax.axis_index("c")
    s = lax.axis_index("s")
    wid = c * _NS + s

    @pl.when(s == 0)
    def _():
        pltpu.sync_copy(zeros.at[pl.ds(0, _NG)], sacc)
        pltpu.sync_copy(zeros.at[pl.ds(0, _NG)], cacc)

    pltpu.sync_copy(ones2, ones_v)
    plsc.subcore_barrier()
    base = wid * _NPW

    def chunk(t, carry):
        off = base + t * _PCH

        @pl.when(off < _N)
        def _():
            pltpu.sync_copy(batch.at[pl.ds(off, _PCH)], idx_b)
            pltpu.sync_copy(xf.at[pl.ds(off, _PCH)], rows)
            pltpu.sync_copy(rows, sacc.at[idx_b], add=True)
            pltpu.sync_copy(ones_v, cacc.at[idx_b], add=True)

        return carry

    lax.fori_loop(0, _NPW // _PCH, chunk, 0)
    plsc.subcore_barrier()

    @pl.when(s == 0)
    def _():
        pltpu.sync_copy(sacc, ssum.at[c])
        pltpu.sync_copy(cacc, cnt.at[c])


_sc_pool = pl.kernel(
    _sc_pool_body,
    out_type=[
        jax.ShapeDtypeStruct((_NC, _NG, _D), jnp.float32),
        jax.ShapeDtypeStruct((_NC, _NG, _D), jnp.float32),
    ],
    mesh=_mesh,
    scratch_types=[
        pltpu.VMEM((_PCH,), jnp.int32),
        pltpu.VMEM((_PCH, _D), jnp.float32),
        pltpu.VMEM((_PCH, _D), jnp.float32),
        pltpu.VMEM_SHARED((_NG, _D), jnp.float32),
        pltpu.VMEM_SHARED((_NG, _D), jnp.float32),
        pltpu.SemaphoreType.DMA,
    ],
)


def _ln_mm(x, g, b, wl, wr):
    mu = jnp.mean(x, axis=-1, keepdims=True)
    xc = x - mu
    var = jnp.mean(xc * xc, axis=-1, keepdims=True)
    ln = xc * lax.rsqrt(var + 1e-5) * g + b
    return (jnp.dot(ln, wl, preferred_element_type=jnp.float32),
            jnp.dot(ln, wr, preferred_element_type=jnp.float32))


def _tc_pre_body(x, g, b, wl, wr, dg, xl_o, hr_o, invd_o):
    xl, hr = _ln_mm(x[...], g[...], b[...], wl[...], wr[...])
    xl_o[...] = xl
    hr_o[...] = hr
    invd_o[...] = 1.0 / jnp.maximum(dg[0] + dg[1], 1.0)


_tc_pre = pl.pallas_call(
    _tc_pre_body,
    out_shape=(
        jax.ShapeDtypeStruct((_N, _D), jnp.float32),
        jax.ShapeDtypeStruct((_N, _D), jnp.float32),
        jax.ShapeDtypeStruct((_N, _D), jnp.float32),
    ),
)


def _agg_of(p, invd):
    return (p[0] + p[1]) * invd[...]


def _tc_mid_body(save_x, has_res, *refs):
    (xl, p, dg, bvec, g2, b2, wl2, wr2), rest = refs[:8], refs[8:]
    if has_res:
        res, rest = rest[0], rest[1:]
    xn = jnp.maximum(xl[...] + _agg_of(p, dg) + bvec[...], 0.0)
    if has_res:
        xn = xn + res[...]
    if save_x:
        x_o, rest = rest[0], rest[1:]
        x_o[...] = xn
    xl_o, hr_o = rest
    xl2, hr2 = _ln_mm(xn, g2[...], b2[...], wl2[...], wr2[...])
    xl_o[...] = xl2
    hr_o[...] = hr2


def _make_mid(save_x, has_res):
    shapes = []
    if save_x:
        shapes.append(jax.ShapeDtypeStruct((_N, _D), jnp.float32))
    shapes.append(jax.ShapeDtypeStruct((_N, _D), jnp.float32))
    shapes.append(jax.ShapeDtypeStruct((_N, _D), jnp.float32))
    return pl.pallas_call(
        functools.partial(_tc_mid_body, save_x, has_res),
        out_shape=tuple(shapes),
    )


_tc_mid_plain = _make_mid(False, False)
_tc_mid_save = _make_mid(True, False)
_tc_mid_res = _make_mid(False, True)


def _tc_last_body(xl, p, dg, bvec, x_o):
    x_o[...] = jnp.maximum(xl[...] + _agg_of(p, dg) + bvec[...], 0.0)


_tc_last = pl.pallas_call(
    _tc_last_body,
    out_shape=jax.ShapeDtypeStruct((_N, _D), jnp.float32),
)


def _tc_out_body(sp, cp, w, b, o):
    pooled = (sp[0] + sp[1]) / jnp.maximum(cp[0] + cp[1], 1.0)
    o[...] = jnp.dot(pooled, w[...], preferred_element_type=jnp.float32) + b[...]


_tc_out = pl.pallas_call(
    _tc_out_body,
    out_shape=jax.ShapeDtypeStruct((_NG, _D), jnp.float32),
)


def kernel(x, edge_index, batch, params):
    ppw = (_EPAD - _E) // _NW
    spad = jnp.broadcast_to(jnp.arange(ppw, dtype=jnp.int32) % 128,
                            (_NW, ppw))
    dpad = jnp.broadcast_to(_N + (jnp.arange(ppw, dtype=jnp.int32) % 120),
                            (_NW, ppw))
    src_p = jnp.concatenate(
        [edge_index[0].reshape(_NW, _E // _NW), spad], axis=1).reshape(-1)
    dst_p = jnp.concatenate(
        [edge_index[1].reshape(_NW, _E // _NW), dpad], axis=1).reshape(-1)
    zeros = jnp.zeros((_NPAD, _D), jnp.float32)
    ones2 = jnp.ones((_PCH, _D), jnp.float32)
    zerosg = jnp.zeros((_NG, _D), jnp.float32)
    ones_n = jnp.ones((_N, _D), jnp.float32)

    dg = _sc_segsum(ones_n, src_p, dst_p, zeros)
    xl, hr, invd = _tc_pre(x, params["gamma0"], params["beta0"],
                           params["Wl0"], params["Wr0"], dg)
    res4 = None
    for i in range(_LAYERS - 1):
        p = _sc_segsum(hr, src_p, dst_p, zeros)
        nxt = (params[f"gamma{i + 1}"], params[f"beta{i + 1}"],
               params[f"Wl{i + 1}"], params[f"Wr{i + 1}"])
        if i == 4:
            res4, xl, hr = _tc_mid_save(xl, p, invd, params[f"b{i}"], *nxt)
        elif i == 6:
            xl, hr = _tc_mid_res(xl, p, invd, params[f"b{i}"], *nxt, res4)
        else:
            xl, hr = _tc_mid_plain(xl, p, invd, params[f"b{i}"], *nxt)
    p = _sc_segsum(hr, src_p, dst_p, zeros)
    xf = _tc_last(xl, p, invd, params[f"b{_LAYERS - 1}"])
    ssum, cnt = _sc_pool(xf, batch, zerosg, ones2)
    return _tc_out(ssum, cnt, params["W_out"], params["b_out"])
